# baseline (speedup 1.0000x reference)
import functools

import jax
import jax.numpy as jnp
from jax import lax
from jax.experimental import pallas as pl
from jax.experimental.pallas import tpu as pltpu
from jax.experimental.pallas import tpu_sc as plsc

N = 10000
E = 320000
E2 = E + N
NPAD = 10112
EPAD = 331776
KT = EPAD // 32
NCH = KT // 128
RPT = NPAD // 16

_mesh = plsc.VectorSubcoreMesh(core_axis_name="c", subcore_axis_name="s")
_sc_params = pltpu.CompilerParams(needs_layout_passes=False,
                                  use_tc_tiling_on_sc=False)


def _iota16():
    return lax.iota(jnp.int32, 16)


def _mm_first_body(x_ref, w_ref, a_ref, o_ref):
    h = jnp.dot(x_ref[...], w_ref[...], preferred_element_type=jnp.float32)
    av = jnp.dot(h, a_ref[...], preferred_element_type=jnp.float32)
    o_ref[:, :128] = h
    o_ref[:, 128:] = av


def _mm_mid_body(p0_ref, p1_ref, b_ref, w_ref, a_ref, o_ref):
    x = jnp.maximum(p0_ref[0] + p1_ref[0] + b_ref[0], 0.0)
    h = jnp.dot(x, w_ref[...], preferred_element_type=jnp.float32)
    av = jnp.dot(h, a_ref[...], preferred_element_type=jnp.float32)
    o_ref[:, :128] = h
    o_ref[:, 128:] = av


def _sum_body(p0_ref, p1_ref, b_ref, o_ref):
    o_ref[...] = p0_ref[0] + p1_ref[0] + b_ref[0]


def _scomb_body(s_ref, o_ref):
    o_ref[...] = s_ref[0] + s_ref[1] + 1e-16


def _scomb(s2):
    return pl.pallas_call(
        _scomb_body,
        in_specs=[pl.BlockSpec((2, NPAD // 128, 128), lambda: (0, 0, 0))],
        out_specs=pl.BlockSpec((NPAD // 128, 128), lambda: (0, 0)),
        out_shape=jax.ShapeDtypeStruct((NPAD // 128, 128), jnp.float32),
    )(s2.reshape(2, NPAD // 128, 128)).reshape(NPAD)


def _mm_first(x_pad, w_pad, a_pad):
    return pl.pallas_call(
        _mm_first_body,
        grid=(NPAD // 128,),
        in_specs=[
            pl.BlockSpec((128, 128), lambda i: (i, 0)),
            pl.BlockSpec((128, 128), lambda i: (0, 0)),
            pl.BlockSpec((128, 128), lambda i: (0, 0)),
        ],
        out_specs=pl.BlockSpec((128, 256), lambda i: (i, 0)),
        out_shape=jax.ShapeDtypeStruct((NPAD, 256), jnp.float32),
    )(x_pad, w_pad, a_pad)


def _mm_mid(acc, b, w_pad, a_pad):
    return pl.pallas_call(
        _mm_mid_body,
        grid=(NPAD // 128,),
        in_specs=[
            pl.BlockSpec((1, 128, 128), lambda i: (0, i, 0)),
            pl.BlockSpec((1, 128, 128), lambda i: (1, i, 0)),
            pl.BlockSpec((1, 128), lambda i: (0, 0)),
            pl.BlockSpec((128, 128), lambda i: (0, 0)),
            pl.BlockSpec((128, 128), lambda i: (0, 0)),
        ],
        out_specs=pl.BlockSpec((128, 256), lambda i: (i, 0)),
        out_shape=jax.ShapeDtypeStruct((NPAD, 256), jnp.float32),
    )(acc, acc, b, w_pad, a_pad)


def _sum_final(acc, b):
    d = acc.shape[-1]
    return pl.pallas_call(
        _sum_body,
        grid=(NPAD // 128,),
        in_specs=[
            pl.BlockSpec((1, 128, d), lambda i: (0, i, 0)),
            pl.BlockSpec((1, 128, d), lambda i: (1, i, 0)),
            pl.BlockSpec((1, d), lambda i: (0, 0)),
        ],
        out_specs=pl.BlockSpec((128, d), lambda i: (i, 0)),
        out_shape=jax.ShapeDtypeStruct((NPAD, d), jnp.float32),
    )(acc, acc, b)


def _sca_body(src_hbm, dst_hbm, asrc_hbm, adst_hbm,
              ex_hbm, s_hbm,
              src_v, dst_v, asrc_v, adst_v, ex_v, rows0, rows1, dstg0,
              dstg1, ztile_v, sc_v, s2d_sh, sem, sem2):
    cid = lax.axis_index("c")
    sid = lax.axis_index("s")
    wid = cid * 16 + sid
    rows = (rows0, rows1)
    dstage = (dstg0, dstg1)
    sems = (sem, sem2)

    pltpu.sync_copy(src_hbm.at[wid], src_v)
    pltpu.sync_copy(dst_hbm.at[wid], dst_v)
    pltpu.sync_copy(asrc_hbm, asrc_v)
    pltpu.sync_copy(adst_hbm, adst_v)

    zv = jnp.zeros((16,), jnp.float32)

    def _zrow(i, _):
        ztile_v[i, :] = zv
        return 0

    lax.fori_loop(0, RPT, _zrow, 0)
    pltpu.sync_copy(ztile_v, s2d_sh.at[pl.ds(sid * RPT, RPT)])

    def _zrow2(i, _):
        rows0[i, :] = zv
        rows1[i, :] = zv
        return 0

    lax.fori_loop(0, 128, _zrow2, 0)
    plsc.subcore_barrier()

    iota = _iota16()
    zcol = jnp.zeros((16,), jnp.int32)

    def _compute(j, b):
        for g in range(8):
            sv = src_v[j, pl.ds(g * 16, 16)]
            dv = dst_v[j, pl.ds(g * 16, 16)]
            e = plsc.load_gather(asrc_v, [sv]) + plsc.load_gather(adst_v, [dv])
            e = jnp.where(e >= 0.0, e, 0.2 * e)
            exv = jnp.exp(e)
            ex_v[j, pl.ds(g * 16, 16)] = exv
            plsc.store_scatter(rows[b], [iota + (g * 16), zcol], exv)
            dstage[b][pl.ds(g * 16, 16)] = dv

    def _chunk(jj, _):
        for b in range(2):
            j = jj * 2 + b

            @pl.when(jj > 0)
            def _():
                pltpu.make_async_copy(rows[b], s2d_sh.at[dstage[b]],
                                      sems[b]).wait()

            _compute(j, b)
            pltpu.async_copy(rows[b], s2d_sh.at[dstage[b]], sems[b],
                             add=True)
        return 0

    lax.fori_loop(0, NCH // 2, _chunk, 0)
    pltpu.make_async_copy(rows[0], s2d_sh.at[dstage[0]], sem).wait()
    _compute(NCH - 1, 0)
    pltpu.async_copy(rows[0], s2d_sh.at[dstage[0]], sem, add=True)
    pltpu.make_async_copy(rows[0], s2d_sh.at[dstage[0]], sem).wait()
    pltpu.make_async_copy(rows[1], s2d_sh.at[dstage[1]], sem2).wait()
    plsc.subcore_barrier()

    pltpu.sync_copy(ex_v, ex_hbm.at[wid])

    pltpu.sync_copy(s2d_sh.at[pl.ds(sid * RPT, RPT)], ztile_v)

    def _comp(i, _):
        base = jnp.minimum(i * 16, RPT - 16)
        sc_v[pl.ds(base, 16)] = plsc.load_gather(ztile_v, [iota + base, zcol])
        return 0

    lax.fori_loop(0, (RPT + 15) // 16, _comp, 0)
    pltpu.sync_copy(sc_v, s_hbm.at[cid, pl.ds(sid * RPT, RPT)])


_sca = pl.kernel(
    _sca_body,
    out_type=(
        jax.ShapeDtypeStruct((32, NCH, 128), jnp.float32),
        jax.ShapeDtypeStruct((2, NPAD), jnp.float32),
    ),
    mesh=_mesh,
    compiler_params=_sc_params,
    scratch_types=(
        pltpu.VMEM((NCH, 128), jnp.int32),
        pltpu.VMEM((NCH, 128), jnp.int32),
        pltpu.VMEM((NPAD,), jnp.float32),
        pltpu.VMEM((NPAD,), jnp.float32),
        pltpu.VMEM((NCH, 128), jnp.float32),
        pltpu.VMEM((128, 16), jnp.float32),
        pltpu.VMEM((128, 16), jnp.float32),
        pltpu.VMEM((128,), jnp.int32),
        pltpu.VMEM((128,), jnp.int32),
        pltpu.VMEM((RPT, 16), jnp.float32),
        pltpu.VMEM((RPT,), jnp.float32),
        pltpu.VMEM_SHARED((NPAD, 16), jnp.float32),
        pltpu.SemaphoreType.DMA,
        pltpu.SemaphoreType.DMA,
    ),
)


CH = 64
NC2 = KT // CH


def _scb_body(d, src16_hbm, dst16_hbm, ex_hbm, s_hbm, h_hbm,
              acc_hbm, al_hbm,
              sb_v, db_v, al_v, s_v, rows0, rows1, sst0, sst1, dst0, dst1,
              acc_sh, sg0, sg1, ss0, ss1):
    cid = lax.axis_index("c")
    sid = lax.axis_index("s")
    wid = cid * 16 + sid
    nk = d // 16
    rows = (rows0, rows1)
    ssts = (sst0, sst1)
    dsts = (dst0, dst1)
    sgs = (sg0, sg1)
    sss = (ss0, ss1)

    pltpu.sync_copy(src16_hbm.at[wid], sb_v)
    pltpu.sync_copy(dst16_hbm.at[wid], db_v)
    pltpu.sync_copy(ex_hbm.at[wid], al_v)
    pltpu.sync_copy(s_hbm, s_v)

    zv = jnp.zeros((16,), jnp.float32)

    def _zrow(i, _):
        for k in range(nk):
            rows0[i, pl.ds(k * 16, 16)] = zv
        return 0

    lax.fori_loop(0, CH, _zrow, 0)
    for r0 in range(0, RPT, CH):
        rr = min(CH, RPT - r0)
        pltpu.sync_copy(rows0.at[pl.ds(0, rr)],
                        acc_sh.at[pl.ds(sid * RPT + r0, rr)])
    plsc.subcore_barrier()

    iota = _iota16()
    mlo = jnp.full((16,), 0xFFFF, jnp.int32)

    def _unpack(j, packed_ref, stage):
        for g in range(2):
            w = packed_ref[pl.ds(j * 32 + g * 16, 16)]
            plsc.store_scatter(stage, [iota * 2 + g * 32],
                               jnp.bitwise_and(w, mlo))
            plsc.store_scatter(stage, [iota * 2 + g * 32 + 1],
                               lax.shift_right_logical(w, 16))

    def _iter(jj, _):
        descs = []
        for b in range(2):
            j = jj * 2 + b

            @pl.when(jj > 0)
            def _():
                pltpu.make_async_copy(rows[b], acc_sh.at[dsts[b]],
                                      sss[b]).wait()

            _unpack(j, sb_v, ssts[b])
            _unpack(j, db_v, dsts[b])
            descs.append(pltpu.async_copy(h_hbm.at[ssts[b]], rows[b],
                                          sgs[b]))
        for b in range(2):
            j = jj * 2 + b

            def _div(g, _):
                sl = pl.ds(g * 16, 16)
                sv = plsc.load_gather(s_v, [dsts[b][sl]])
                al_v[j, sl] = al_v[j, sl] / sv
                return 0

            lax.fori_loop(0, CH // 16, _div, 0)
            descs[b].wait()

            def _grp(g, _):
                av = al_v[j, pl.ds(g * 16, 16)]
                for e16 in range(16):
                    e = g * 16 + e16
                    splat = jnp.broadcast_to(av[e16], (16,))
                    for k in range(nk):
                        rows[b][e, pl.ds(k * 16, 16)] = (
                            rows[b][e, pl.ds(k * 16, 16)] * splat)
                return 0

            lax.fori_loop(0, CH // 16, _grp, 0)
            pltpu.async_copy(rows[b], acc_sh.at[dsts[b]], sss[b], add=True)
        return 0

    lax.fori_loop(0, NC2 // 2, _iter, 0)
    for b in range(2):
        pltpu.make_async_copy(rows[b], acc_sh.at[dsts[b]], sss[b]).wait()
    plsc.subcore_barrier()

    pltpu.sync_copy(acc_sh.at[pl.ds(sid * RPT, RPT)],
                    acc_hbm.at[cid, pl.ds(sid * RPT, RPT)])
    pltpu.sync_copy(al_v, al_hbm.at[wid])


@functools.cache
def _make_scb(d):
    return pl.kernel(
        functools.partial(_scb_body, d),
        out_type=(
            jax.ShapeDtypeStruct((2, NPAD, d), jnp.float32),
            jax.ShapeDtypeStruct((32, NC2, CH), jnp.float32),
        ),
        mesh=_mesh,
        compiler_params=_sc_params,
        scratch_types=(
            pltpu.VMEM((KT // 2,), jnp.int32),
            pltpu.VMEM((KT // 2,), jnp.int32),
            pltpu.VMEM((NC2, CH), jnp.float32),
            pltpu.VMEM((NPAD,), jnp.float32),
            pltpu.VMEM((CH, d), jnp.float32),
            pltpu.VMEM((CH, d), jnp.float32),
            pltpu.VMEM((CH,), jnp.int32),
            pltpu.VMEM((CH,), jnp.int32),
            pltpu.VMEM((CH,), jnp.int32),
            pltpu.VMEM((CH,), jnp.int32),
            pltpu.VMEM_SHARED((NPAD, d), jnp.float32),
            pltpu.SemaphoreType.DMA,
            pltpu.SemaphoreType.DMA,
            pltpu.SemaphoreType.DMA,
            pltpu.SemaphoreType.DMA,
        ),
    )


def _pack_attn(a_s, a_d):
    a = jnp.stack([a_s, a_d], axis=1)
    return jnp.pad(a, ((0, 128 - a.shape[0]), (0, 126)))


def kernel(x, edge_index, W1, as1, ad1, b1, W2, as2, ad2, b2,
           W3, as3, ad3, b3):
    loops = jnp.arange(N, dtype=edge_index.dtype)
    ei2 = jnp.concatenate(
        [edge_index, jnp.stack([loops, loops], axis=0)], axis=1)

    pad_s = jnp.full((EPAD - E2,), N, jnp.int32)
    pad_d = N + jnp.arange(EPAD - E2, dtype=jnp.int32) % (NPAD - N)
    src_f = jnp.concatenate([ei2[0], pad_s])
    dst_f = jnp.concatenate([ei2[1], pad_d])
    src = src_f.reshape(32, NCH, 128)
    dst = dst_f.reshape(32, NCH, 128)

    def _pack16(a):
        return lax.bitcast_convert_type(
            a.astype(jnp.int16).reshape(EPAD // 2, 2),
            jnp.int32).reshape(32, KT // 2)

    src16 = _pack16(src_f)
    dst16 = _pack16(dst_f)

    x_pad = jnp.pad(x, ((0, NPAD - N), (0, 0)))
    w1p = W1
    w2p = W2
    w3p = jnp.pad(W3, ((0, 0), (0, 128 - W3.shape[1])))
    a1p = _pack_attn(as1, ad1)
    a2p = _pack_attn(as2, ad2)
    a3p = _pack_attn(as3, ad3)

    def _edge_layer(y, d):
        h = y[:, :d]
        asrc = y[:, 128]
        adst = y[:, 129]
        ex, s = _sca(src, dst, asrc, adst)
        sc = _scomb(s)
        acc, alpha = _make_scb(d)(src16, dst16, ex.reshape(32, NC2, CH),
                                  sc, h)
        return acc, alpha.reshape(EPAD)

    y1 = _mm_first(x_pad, w1p, a1p)
    acc1, alpha = _edge_layer(y1, 128)
    y2 = _mm_mid(acc1, b1[None, :], w2p, a2p)
    acc2, _ = _edge_layer(y2, 128)
    y3 = _mm_mid(acc2, b2[None, :], w3p, a3p)
    acc3, _ = _edge_layer(y3, 64)
    h3 = _sum_final(acc3, b3[None, :])

    return (ei2, alpha[:E2], h3[:N])


# speedup vs baseline: 1.0205x; 1.0205x over previous
import functools

import jax
import jax.numpy as jnp
from jax import lax
from jax.experimental import pallas as pl
from jax.experimental.pallas import tpu as pltpu
from jax.experimental.pallas import tpu_sc as plsc

N = 10000
E = 320000
E2 = E + N
NPAD = 10112
EPAD = 331776
KT = EPAD // 32
NCH = KT // 128
RPT = NPAD // 16

_mesh = plsc.VectorSubcoreMesh(core_axis_name="c", subcore_axis_name="s")
_sc_params = pltpu.CompilerParams(needs_layout_passes=False,
                                  use_tc_tiling_on_sc=False)


def _iota16():
    return lax.iota(jnp.int32, 16)


def _mm_first_body(x_ref, w_ref, a_ref, o_ref):
    h = jnp.dot(x_ref[...], w_ref[...], preferred_element_type=jnp.float32)
    av = jnp.dot(h, a_ref[...], preferred_element_type=jnp.float32)
    o_ref[:, :128] = h
    o_ref[:, 128:] = av


def _mm_mid_body(p0_ref, p1_ref, b_ref, w_ref, a_ref, o_ref):
    x = jnp.maximum(p0_ref[0] + p1_ref[0] + b_ref[0], 0.0)
    h = jnp.dot(x, w_ref[...], preferred_element_type=jnp.float32)
    av = jnp.dot(h, a_ref[...], preferred_element_type=jnp.float32)
    o_ref[:, :128] = h
    o_ref[:, 128:] = av


def _sum_body(p0_ref, p1_ref, b_ref, o_ref):
    o_ref[...] = p0_ref[0] + p1_ref[0] + b_ref[0]


def _scomb_body(s_ref, o_ref):
    o_ref[...] = s_ref[0] + s_ref[1] + 1e-16


def _scomb(s2):
    return pl.pallas_call(
        _scomb_body,
        in_specs=[pl.BlockSpec((2, NPAD // 128, 128), lambda: (0, 0, 0))],
        out_specs=pl.BlockSpec((NPAD // 128, 128), lambda: (0, 0)),
        out_shape=jax.ShapeDtypeStruct((NPAD // 128, 128), jnp.float32),
    )(s2.reshape(2, NPAD // 128, 128)).reshape(NPAD)


def _mm_first(x_pad, w_pad, a_pad):
    return pl.pallas_call(
        _mm_first_body,
        grid=(NPAD // 128,),
        in_specs=[
            pl.BlockSpec((128, 128), lambda i: (i, 0)),
            pl.BlockSpec((128, 128), lambda i: (0, 0)),
            pl.BlockSpec((128, 128), lambda i: (0, 0)),
        ],
        out_specs=pl.BlockSpec((128, 256), lambda i: (i, 0)),
        out_shape=jax.ShapeDtypeStruct((NPAD, 256), jnp.float32),
    )(x_pad, w_pad, a_pad)


def _mm_mid(acc, b, w_pad, a_pad):
    return pl.pallas_call(
        _mm_mid_body,
        grid=(NPAD // 128,),
        in_specs=[
            pl.BlockSpec((1, 128, 128), lambda i: (0, i, 0)),
            pl.BlockSpec((1, 128, 128), lambda i: (1, i, 0)),
            pl.BlockSpec((1, 128), lambda i: (0, 0)),
            pl.BlockSpec((128, 128), lambda i: (0, 0)),
            pl.BlockSpec((128, 128), lambda i: (0, 0)),
        ],
        out_specs=pl.BlockSpec((128, 256), lambda i: (i, 0)),
        out_shape=jax.ShapeDtypeStruct((NPAD, 256), jnp.float32),
    )(acc, acc, b, w_pad, a_pad)


def _sum_final(acc, b):
    d = acc.shape[-1]
    return pl.pallas_call(
        _sum_body,
        grid=(NPAD // 128,),
        in_specs=[
            pl.BlockSpec((1, 128, d), lambda i: (0, i, 0)),
            pl.BlockSpec((1, 128, d), lambda i: (1, i, 0)),
            pl.BlockSpec((1, d), lambda i: (0, 0)),
        ],
        out_specs=pl.BlockSpec((128, d), lambda i: (i, 0)),
        out_shape=jax.ShapeDtypeStruct((NPAD, d), jnp.float32),
    )(acc, acc, b)


def _sca_body(src_hbm, dst_hbm, asrc_hbm, adst_hbm,
              ex_hbm, s_hbm,
              src_v, dst_v, asrc_v, adst_v, ex_v, rows0, rows1, dstg0,
              dstg1, ztile_v, sc_v, s2d_sh, sem, sem2):
    cid = lax.axis_index("c")
    sid = lax.axis_index("s")
    wid = cid * 16 + sid
    rows = (rows0, rows1)
    dstage = (dstg0, dstg1)
    sems = (sem, sem2)

    pltpu.sync_copy(src_hbm.at[wid], src_v)
    pltpu.sync_copy(dst_hbm.at[wid], dst_v)
    pltpu.sync_copy(asrc_hbm, asrc_v)
    pltpu.sync_copy(adst_hbm, adst_v)

    zv = jnp.zeros((16,), jnp.float32)

    def _zrow(i, _):
        ztile_v[i, :] = zv
        return 0

    lax.fori_loop(0, RPT, _zrow, 0)
    pltpu.sync_copy(ztile_v, s2d_sh.at[pl.ds(sid * RPT, RPT)])

    def _zrow2(i, _):
        rows0[i, :] = zv
        rows1[i, :] = zv
        return 0

    lax.fori_loop(0, 128, _zrow2, 0)
    plsc.subcore_barrier()

    iota = _iota16()
    zcol = jnp.zeros((16,), jnp.int32)

    def _compute(j, b):
        for g in range(8):
            sv = src_v[j, pl.ds(g * 16, 16)]
            dv = dst_v[j, pl.ds(g * 16, 16)]
            e = plsc.load_gather(asrc_v, [sv]) + plsc.load_gather(adst_v, [dv])
            e = jnp.where(e >= 0.0, e, 0.2 * e)
            exv = jnp.exp(e)
            ex_v[j, pl.ds(g * 16, 16)] = exv
            plsc.store_scatter(rows[b], [iota + (g * 16), zcol], exv)
            dstage[b][pl.ds(g * 16, 16)] = dv

    def _chunk(jj, _):
        for b in range(2):
            j = jj * 2 + b

            @pl.when(jj > 0)
            def _():
                pltpu.make_async_copy(rows[b], s2d_sh.at[dstage[b]],
                                      sems[b]).wait()

            _compute(j, b)
            pltpu.async_copy(rows[b], s2d_sh.at[dstage[b]], sems[b],
                             add=True)
        return 0

    lax.fori_loop(0, NCH // 2, _chunk, 0)
    pltpu.make_async_copy(rows[0], s2d_sh.at[dstage[0]], sem).wait()
    _compute(NCH - 1, 0)
    pltpu.async_copy(rows[0], s2d_sh.at[dstage[0]], sem, add=True)
    pltpu.make_async_copy(rows[0], s2d_sh.at[dstage[0]], sem).wait()
    pltpu.make_async_copy(rows[1], s2d_sh.at[dstage[1]], sem2).wait()
    plsc.subcore_barrier()

    pltpu.sync_copy(ex_v, ex_hbm.at[wid])

    pltpu.sync_copy(s2d_sh.at[pl.ds(sid * RPT, RPT)], ztile_v)

    def _comp(i, _):
        base = jnp.minimum(i * 16, RPT - 16)
        sc_v[pl.ds(base, 16)] = plsc.load_gather(ztile_v, [iota + base, zcol])
        return 0

    lax.fori_loop(0, (RPT + 15) // 16, _comp, 0)
    pltpu.sync_copy(sc_v, s_hbm.at[cid, pl.ds(sid * RPT, RPT)])


_sca = pl.kernel(
    _sca_body,
    out_type=(
        jax.ShapeDtypeStruct((32, NCH, 128), jnp.float32),
        jax.ShapeDtypeStruct((2, NPAD), jnp.float32),
    ),
    mesh=_mesh,
    compiler_params=_sc_params,
    scratch_types=(
        pltpu.VMEM((NCH, 128), jnp.int32),
        pltpu.VMEM((NCH, 128), jnp.int32),
        pltpu.VMEM((NPAD,), jnp.float32),
        pltpu.VMEM((NPAD,), jnp.float32),
        pltpu.VMEM((NCH, 128), jnp.float32),
        pltpu.VMEM((128, 16), jnp.float32),
        pltpu.VMEM((128, 16), jnp.float32),
        pltpu.VMEM((128,), jnp.int32),
        pltpu.VMEM((128,), jnp.int32),
        pltpu.VMEM((RPT, 16), jnp.float32),
        pltpu.VMEM((RPT,), jnp.float32),
        pltpu.VMEM_SHARED((NPAD, 16), jnp.float32),
        pltpu.SemaphoreType.DMA,
        pltpu.SemaphoreType.DMA,
    ),
)


CH = 64
NC2 = KT // CH


def _scb_body(d, src16_hbm, dst16_hbm, ex_hbm, s_hbm, h_hbm,
              acc_hbm, al_hbm,
              sb_v, db_v, al_v, s_v, rows0, rows1, sst0, sst1, dst0, dst1,
              acc_sh, sg0, sg1, ss0, ss1):
    cid = lax.axis_index("c")
    sid = lax.axis_index("s")
    wid = cid * 16 + sid
    nk = d // 16
    rows = (rows0, rows1)
    ssts = (sst0, sst1)
    dsts = (dst0, dst1)
    sgs = (sg0, sg1)
    sss = (ss0, ss1)

    pltpu.sync_copy(src16_hbm.at[wid], sb_v)
    pltpu.sync_copy(dst16_hbm.at[wid], db_v)
    pltpu.sync_copy(ex_hbm.at[wid], al_v)
    pltpu.sync_copy(s_hbm, s_v)

    zv = jnp.zeros((16,), jnp.float32)

    def _zrow(i, _):
        for k in range(nk):
            rows0[i, pl.ds(k * 16, 16)] = zv
        return 0

    lax.fori_loop(0, CH, _zrow, 0)
    for r0 in range(0, RPT, CH):
        rr = min(CH, RPT - r0)
        pltpu.sync_copy(rows0.at[pl.ds(0, rr)],
                        acc_sh.at[pl.ds(sid * RPT + r0, rr)])
    plsc.subcore_barrier()

    iota = _iota16()
    mlo = jnp.full((16,), 0xFFFF, jnp.int32)

    def _unpack(j, packed_ref, stage):
        for g in range(2):
            w = packed_ref[pl.ds(j * 32 + g * 16, 16)]
            plsc.store_scatter(stage, [iota * 2 + g * 32],
                               jnp.bitwise_and(w, mlo))
            plsc.store_scatter(stage, [iota * 2 + g * 32 + 1],
                               lax.shift_right_logical(w, 16))

    def _iter(jj, _):
        descs = []
        for b in range(2):
            j = jj * 2 + b

            @pl.when(jj > 0)
            def _():
                pltpu.make_async_copy(rows[b], acc_sh.at[dsts[b]],
                                      sss[b]).wait()

            _unpack(j, sb_v, ssts[b])
            _unpack(j, db_v, dsts[b])
            descs.append(pltpu.async_copy(h_hbm.at[ssts[b]], rows[b],
                                          sgs[b]))
        for b in range(2):
            j = jj * 2 + b

            def _div(g, _):
                sl = pl.ds(g * 16, 16)
                sv = plsc.load_gather(s_v, [dsts[b][sl]])
                al_v[j, sl] = al_v[j, sl] / sv
                return 0

            lax.fori_loop(0, CH // 16, _div, 0)
            descs[b].wait()

            jv = jnp.full((16,), j, jnp.int32)

            def _grp(g, _):
                for e16 in range(16):
                    e = g * 16 + e16
                    splat = plsc.load_gather(
                        al_v, [jv, jnp.full((16,), e16, jnp.int32) + g * 16])
                    for k in range(nk):
                        rows[b][e, pl.ds(k * 16, 16)] = (
                            rows[b][e, pl.ds(k * 16, 16)] * splat)
                return 0

            lax.fori_loop(0, CH // 16, _grp, 0)
            pltpu.async_copy(rows[b], acc_sh.at[dsts[b]], sss[b], add=True)
        return 0

    lax.fori_loop(0, NC2 // 2, _iter, 0)
    for b in range(2):
        pltpu.make_async_copy(rows[b], acc_sh.at[dsts[b]], sss[b]).wait()
    plsc.subcore_barrier()

    pltpu.sync_copy(acc_sh.at[pl.ds(sid * RPT, RPT)],
                    acc_hbm.at[cid, pl.ds(sid * RPT, RPT)])
    pltpu.sync_copy(al_v, al_hbm.at[wid])


@functools.cache
def _make_scb(d):
    return pl.kernel(
        functools.partial(_scb_body, d),
        out_type=(
            jax.ShapeDtypeStruct((2, NPAD, d), jnp.float32),
            jax.ShapeDtypeStruct((32, NC2, CH), jnp.float32),
        ),
        mesh=_mesh,
        compiler_params=_sc_params,
        scratch_types=(
            pltpu.VMEM((KT // 2,), jnp.int32),
            pltpu.VMEM((KT // 2,), jnp.int32),
            pltpu.VMEM((NC2, CH), jnp.float32),
            pltpu.VMEM((NPAD,), jnp.float32),
            pltpu.VMEM((CH, d), jnp.float32),
            pltpu.VMEM((CH, d), jnp.float32),
            pltpu.VMEM((CH,), jnp.int32),
            pltpu.VMEM((CH,), jnp.int32),
            pltpu.VMEM((CH,), jnp.int32),
            pltpu.VMEM((CH,), jnp.int32),
            pltpu.VMEM_SHARED((NPAD, d), jnp.float32),
            pltpu.SemaphoreType.DMA,
            pltpu.SemaphoreType.DMA,
            pltpu.SemaphoreType.DMA,
            pltpu.SemaphoreType.DMA,
        ),
    )


def _pack_attn(a_s, a_d):
    a = jnp.stack([a_s, a_d], axis=1)
    return jnp.pad(a, ((0, 128 - a.shape[0]), (0, 126)))


def kernel(x, edge_index, W1, as1, ad1, b1, W2, as2, ad2, b2,
           W3, as3, ad3, b3):
    loops = jnp.arange(N, dtype=edge_index.dtype)
    ei2 = jnp.concatenate(
        [edge_index, jnp.stack([loops, loops], axis=0)], axis=1)

    pad_s = jnp.full((EPAD - E2,), N, jnp.int32)
    pad_d = N + jnp.arange(EPAD - E2, dtype=jnp.int32) % (NPAD - N)
    src_f = jnp.concatenate([ei2[0], pad_s])
    dst_f = jnp.concatenate([ei2[1], pad_d])
    src = src_f.reshape(32, NCH, 128)
    dst = dst_f.reshape(32, NCH, 128)

    def _pack16(a):
        return lax.bitcast_convert_type(
            a.astype(jnp.int16).reshape(EPAD // 2, 2),
            jnp.int32).reshape(32, KT // 2)

    src16 = _pack16(src_f)
    dst16 = _pack16(dst_f)

    x_pad = jnp.pad(x, ((0, NPAD - N), (0, 0)))
    w1p = W1
    w2p = W2
    w3p = jnp.pad(W3, ((0, 0), (0, 128 - W3.shape[1])))
    a1p = _pack_attn(as1, ad1)
    a2p = _pack_attn(as2, ad2)
    a3p = _pack_attn(as3, ad3)

    def _edge_layer(y, d):
        h = y[:, :d]
        asrc = y[:, 128]
        adst = y[:, 129]
        ex, s = _sca(src, dst, asrc, adst)
        sc = _scomb(s)
        acc, alpha = _make_scb(d)(src16, dst16, ex.reshape(32, NC2, CH),
                                  sc, h)
        return acc, alpha.reshape(EPAD)

    y1 = _mm_first(x_pad, w1p, a1p)
    acc1, alpha = _edge_layer(y1, 128)
    y2 = _mm_mid(acc1, b1[None, :], w2p, a2p)
    acc2, _ = _edge_layer(y2, 128)
    y3 = _mm_mid(acc2, b2[None, :], w3p, a3p)
    acc3, _ = _edge_layer(y3, 64)
    h3 = _sum_final(acc3, b3[None, :])

    return (ei2, alpha[:E2], h3[:N])


# speedup vs baseline: 1.0383x; 1.0174x over previous
import functools

import jax
import jax.numpy as jnp
from jax import lax
from jax.experimental import pallas as pl
from jax.experimental.pallas import tpu as pltpu
from jax.experimental.pallas import tpu_sc as plsc

N = 10000
E = 320000
E2 = E + N
NPAD = 10112
EPAD = 331776
KT = EPAD // 32
NCH = KT // 128
RPT = NPAD // 16

_mesh = plsc.VectorSubcoreMesh(core_axis_name="c", subcore_axis_name="s")
_sc_params = pltpu.CompilerParams(needs_layout_passes=False,
                                  use_tc_tiling_on_sc=False)


def _iota16():
    return lax.iota(jnp.int32, 16)


def _mm_first_body(x_ref, w_ref, a_ref, o_ref):
    h = jnp.dot(x_ref[...], w_ref[...], preferred_element_type=jnp.float32)
    av = jnp.dot(h, a_ref[...], preferred_element_type=jnp.float32)
    o_ref[:, :128] = h
    o_ref[:, 128:] = av


def _mm_mid_body(p0_ref, p1_ref, b_ref, w_ref, a_ref, o_ref):
    x = jnp.maximum(p0_ref[0] + p1_ref[0] + b_ref[0], 0.0)
    h = jnp.dot(x, w_ref[...], preferred_element_type=jnp.float32)
    av = jnp.dot(h, a_ref[...], preferred_element_type=jnp.float32)
    o_ref[:, :128] = h
    o_ref[:, 128:] = av


def _sum_body(p0_ref, p1_ref, b_ref, o_ref):
    o_ref[...] = p0_ref[0] + p1_ref[0] + b_ref[0]


def _scomb_body(s_ref, o_ref):
    o_ref[...] = s_ref[0] + s_ref[1] + 1e-16


def _scomb(s2):
    return pl.pallas_call(
        _scomb_body,
        in_specs=[pl.BlockSpec((2, NPAD // 128, 128), lambda: (0, 0, 0))],
        out_specs=pl.BlockSpec((NPAD // 128, 128), lambda: (0, 0)),
        out_shape=jax.ShapeDtypeStruct((NPAD // 128, 128), jnp.float32),
    )(s2.reshape(2, NPAD // 128, 128)).reshape(NPAD)


def _mm_first(x_pad, w_pad, a_pad):
    return pl.pallas_call(
        _mm_first_body,
        grid=(NPAD // 128,),
        in_specs=[
            pl.BlockSpec((128, 128), lambda i: (i, 0)),
            pl.BlockSpec((128, 128), lambda i: (0, 0)),
            pl.BlockSpec((128, 128), lambda i: (0, 0)),
        ],
        out_specs=pl.BlockSpec((128, 256), lambda i: (i, 0)),
        out_shape=jax.ShapeDtypeStruct((NPAD, 256), jnp.float32),
    )(x_pad, w_pad, a_pad)


def _mm_mid(acc, b, w_pad, a_pad):
    return pl.pallas_call(
        _mm_mid_body,
        grid=(NPAD // 128,),
        in_specs=[
            pl.BlockSpec((1, 128, 128), lambda i: (0, i, 0)),
            pl.BlockSpec((1, 128, 128), lambda i: (1, i, 0)),
            pl.BlockSpec((1, 128), lambda i: (0, 0)),
            pl.BlockSpec((128, 128), lambda i: (0, 0)),
            pl.BlockSpec((128, 128), lambda i: (0, 0)),
        ],
        out_specs=pl.BlockSpec((128, 256), lambda i: (i, 0)),
        out_shape=jax.ShapeDtypeStruct((NPAD, 256), jnp.float32),
    )(acc, acc, b, w_pad, a_pad)


def _sum_final(acc, b):
    d = acc.shape[-1]
    return pl.pallas_call(
        _sum_body,
        grid=(NPAD // 128,),
        in_specs=[
            pl.BlockSpec((1, 128, d), lambda i: (0, i, 0)),
            pl.BlockSpec((1, 128, d), lambda i: (1, i, 0)),
            pl.BlockSpec((1, d), lambda i: (0, 0)),
        ],
        out_specs=pl.BlockSpec((128, d), lambda i: (i, 0)),
        out_shape=jax.ShapeDtypeStruct((NPAD, d), jnp.float32),
    )(acc, acc, b)


def _sca_body(src_hbm, dst_hbm, asrc_hbm, adst_hbm,
              ex_hbm, s_hbm,
              src_v, dst_v, asrc_v, adst_v, ex_v, rows0, rows1, dstg0,
              dstg1, ztile_v, sc_v, s2d_sh, sem, sem2):
    cid = lax.axis_index("c")
    sid = lax.axis_index("s")
    wid = cid * 16 + sid
    rows = (rows0, rows1)
    dstage = (dstg0, dstg1)
    sems = (sem, sem2)

    pltpu.sync_copy(src_hbm.at[wid], src_v)
    pltpu.sync_copy(dst_hbm.at[wid], dst_v)
    pltpu.sync_copy(asrc_hbm, asrc_v)
    pltpu.sync_copy(adst_hbm, adst_v)

    zv = jnp.zeros((16,), jnp.float32)

    def _zrow(i, _):
        ztile_v[i, :] = zv
        return 0

    lax.fori_loop(0, RPT, _zrow, 0)
    pltpu.sync_copy(ztile_v, s2d_sh.at[pl.ds(sid * RPT, RPT)])

    def _zrow2(i, _):
        rows0[i, :] = zv
        rows1[i, :] = zv
        return 0

    lax.fori_loop(0, 128, _zrow2, 0)
    plsc.subcore_barrier()

    iota = _iota16()
    zcol = jnp.zeros((16,), jnp.int32)

    def _compute(j, b):
        for g in range(8):
            sv = src_v[j, pl.ds(g * 16, 16)]
            dv = dst_v[j, pl.ds(g * 16, 16)]
            e = plsc.load_gather(asrc_v, [sv]) + plsc.load_gather(adst_v, [dv])
            e = jnp.where(e >= 0.0, e, 0.2 * e)
            exv = jnp.exp(e)
            ex_v[j, pl.ds(g * 16, 16)] = exv
            plsc.store_scatter(rows[b], [iota + (g * 16), zcol], exv)
            dstage[b][pl.ds(g * 16, 16)] = dv

    def _chunk(jj, _):
        for b in range(2):
            j = jj * 2 + b

            @pl.when(jj > 0)
            def _():
                pltpu.make_async_copy(rows[b], s2d_sh.at[dstage[b]],
                                      sems[b]).wait()

            _compute(j, b)
            pltpu.async_copy(rows[b], s2d_sh.at[dstage[b]], sems[b],
                             add=True)
        return 0

    lax.fori_loop(0, NCH // 2, _chunk, 0)
    pltpu.make_async_copy(rows[0], s2d_sh.at[dstage[0]], sem).wait()
    _compute(NCH - 1, 0)
    pltpu.async_copy(rows[0], s2d_sh.at[dstage[0]], sem, add=True)
    pltpu.make_async_copy(rows[0], s2d_sh.at[dstage[0]], sem).wait()
    pltpu.make_async_copy(rows[1], s2d_sh.at[dstage[1]], sem2).wait()
    plsc.subcore_barrier()

    pltpu.sync_copy(ex_v, ex_hbm.at[wid])

    pltpu.sync_copy(s2d_sh.at[pl.ds(sid * RPT, RPT)], ztile_v)

    def _comp(i, _):
        base = jnp.minimum(i * 16, RPT - 16)
        sc_v[pl.ds(base, 16)] = plsc.load_gather(ztile_v, [iota + base, zcol])
        return 0

    lax.fori_loop(0, (RPT + 15) // 16, _comp, 0)
    pltpu.sync_copy(sc_v, s_hbm.at[cid, pl.ds(sid * RPT, RPT)])


_sca = pl.kernel(
    _sca_body,
    out_type=(
        jax.ShapeDtypeStruct((32, NCH, 128), jnp.float32),
        jax.ShapeDtypeStruct((2, NPAD), jnp.float32),
    ),
    mesh=_mesh,
    compiler_params=_sc_params,
    scratch_types=(
        pltpu.VMEM((NCH, 128), jnp.int32),
        pltpu.VMEM((NCH, 128), jnp.int32),
        pltpu.VMEM((NPAD,), jnp.float32),
        pltpu.VMEM((NPAD,), jnp.float32),
        pltpu.VMEM((NCH, 128), jnp.float32),
        pltpu.VMEM((128, 16), jnp.float32),
        pltpu.VMEM((128, 16), jnp.float32),
        pltpu.VMEM((128,), jnp.int32),
        pltpu.VMEM((128,), jnp.int32),
        pltpu.VMEM((RPT, 16), jnp.float32),
        pltpu.VMEM((RPT,), jnp.float32),
        pltpu.VMEM_SHARED((NPAD, 16), jnp.float32),
        pltpu.SemaphoreType.DMA,
        pltpu.SemaphoreType.DMA,
    ),
)


def _scb_body(d, ch, nbuf, src16_hbm, dst16_hbm, ex_hbm, s_hbm, h_hbm,
              acc_hbm, al_hbm, *scr):
    nc2 = KT // ch
    sb_v, db_v, al_v, s_v = scr[:4]
    rows = scr[4:4 + nbuf]
    ssts = scr[4 + nbuf:4 + 2 * nbuf]
    dsts = scr[4 + 2 * nbuf:4 + 3 * nbuf]
    acc_sh = scr[4 + 3 * nbuf]
    sgs = scr[5 + 3 * nbuf:5 + 4 * nbuf]
    sss = scr[5 + 4 * nbuf:5 + 5 * nbuf]
    cid = lax.axis_index("c")
    sid = lax.axis_index("s")
    wid = cid * 16 + sid
    nk = d // 16

    pltpu.sync_copy(src16_hbm.at[wid], sb_v)
    pltpu.sync_copy(dst16_hbm.at[wid], db_v)
    pltpu.sync_copy(ex_hbm.at[wid], al_v)
    pltpu.sync_copy(s_hbm, s_v)

    zv = jnp.zeros((16,), jnp.float32)

    def _zrow(i, _):
        for k in range(nk):
            rows[0][i, pl.ds(k * 16, 16)] = zv
        return 0

    lax.fori_loop(0, ch, _zrow, 0)
    for r0 in range(0, RPT, ch):
        rr = min(ch, RPT - r0)
        pltpu.sync_copy(rows[0].at[pl.ds(0, rr)],
                        acc_sh.at[pl.ds(sid * RPT + r0, rr)])
    plsc.subcore_barrier()

    iota = _iota16()
    mlo = jnp.full((16,), 0xFFFF, jnp.int32)

    def _unpack(j, packed_ref, stage):
        for g in range(ch // 32):
            w = packed_ref[pl.ds(j * (ch // 2) + g * 16, 16)]
            plsc.store_scatter(stage, [iota * 2 + g * 32],
                               jnp.bitwise_and(w, mlo))
            plsc.store_scatter(stage, [iota * 2 + g * 32 + 1],
                               lax.shift_right_logical(w, 16))

    def _iter(jj, _):
        descs = []
        for b in range(nbuf):
            j = jj * nbuf + b

            @pl.when(jj > 0)
            def _():
                pltpu.make_async_copy(rows[b], acc_sh.at[dsts[b]],
                                      sss[b]).wait()

            _unpack(j, sb_v, ssts[b])
            _unpack(j, db_v, dsts[b])
            descs.append(pltpu.async_copy(h_hbm.at[ssts[b]], rows[b],
                                          sgs[b]))
        for b in range(nbuf):
            j = jj * nbuf + b

            def _div(g, _):
                sl = pl.ds(g * 16, 16)
                sv = plsc.load_gather(s_v, [dsts[b][sl]])
                al_v[j, sl] = al_v[j, sl] / sv
                return 0

            lax.fori_loop(0, ch // 16, _div, 0)
            descs[b].wait()

            jv = jnp.full((16,), j, jnp.int32)

            def _grp(g, _):
                base = g * 16

                def _one(e16):
                    e = base + e16
                    splat = plsc.load_gather(
                        al_v, [jv, jnp.full((16,), e16, jnp.int32) + base])
                    for k in range(nk):
                        rows[b][e, pl.ds(k * 16, 16)] = (
                            rows[b][e, pl.ds(k * 16, 16)] * splat)

                for e16 in range(16):
                    _one(e16)
                return 0

            lax.fori_loop(0, ch // 16, _grp, 0)
            pltpu.async_copy(rows[b], acc_sh.at[dsts[b]], sss[b], add=True)
        return 0

    lax.fori_loop(0, nc2 // nbuf, _iter, 0)
    for b in range(nbuf):
        pltpu.make_async_copy(rows[b], acc_sh.at[dsts[b]], sss[b]).wait()
    plsc.subcore_barrier()

    pltpu.sync_copy(acc_sh.at[pl.ds(sid * RPT, RPT)],
                    acc_hbm.at[cid, pl.ds(sid * RPT, RPT)])
    pltpu.sync_copy(al_v, al_hbm.at[wid])


@functools.cache
def _make_scb(d, ch, nbuf):
    nc2 = KT // ch
    return pl.kernel(
        functools.partial(_scb_body, d, ch, nbuf),
        out_type=(
            jax.ShapeDtypeStruct((2, NPAD, d), jnp.float32),
            jax.ShapeDtypeStruct((32, nc2, ch), jnp.float32),
        ),
        mesh=_mesh,
        compiler_params=_sc_params,
        scratch_types=(
            pltpu.VMEM((KT // 2,), jnp.int32),
            pltpu.VMEM((KT // 2,), jnp.int32),
            pltpu.VMEM((nc2, ch), jnp.float32),
            pltpu.VMEM((NPAD,), jnp.float32),
            *[pltpu.VMEM((ch, d), jnp.float32) for _ in range(nbuf)],
            *[pltpu.VMEM((ch,), jnp.int32) for _ in range(2 * nbuf)],
            pltpu.VMEM_SHARED((NPAD, d), jnp.float32),
            *[pltpu.SemaphoreType.DMA for _ in range(2 * nbuf)],
        ),
    )


def _pack_attn(a_s, a_d):
    a = jnp.stack([a_s, a_d], axis=1)
    return jnp.pad(a, ((0, 128 - a.shape[0]), (0, 126)))


def kernel(x, edge_index, W1, as1, ad1, b1, W2, as2, ad2, b2,
           W3, as3, ad3, b3):
    loops = jnp.arange(N, dtype=edge_index.dtype)
    ei2 = jnp.concatenate(
        [edge_index, jnp.stack([loops, loops], axis=0)], axis=1)

    pad_s = jnp.full((EPAD - E2,), N, jnp.int32)
    pad_d = N + jnp.arange(EPAD - E2, dtype=jnp.int32) % (NPAD - N)
    src_f = jnp.concatenate([ei2[0], pad_s])
    dst_f = jnp.concatenate([ei2[1], pad_d])
    src = src_f.reshape(32, NCH, 128)
    dst = dst_f.reshape(32, NCH, 128)

    def _pack16(a):
        return lax.bitcast_convert_type(
            a.astype(jnp.int16).reshape(EPAD // 2, 2),
            jnp.int32).reshape(32, KT // 2)

    src16 = _pack16(src_f)
    dst16 = _pack16(dst_f)

    x_pad = jnp.pad(x, ((0, NPAD - N), (0, 0)))
    w1p = W1
    w2p = W2
    w3p = jnp.pad(W3, ((0, 0), (0, 128 - W3.shape[1])))
    a1p = _pack_attn(as1, ad1)
    a2p = _pack_attn(as2, ad2)
    a3p = _pack_attn(as3, ad3)

    def _edge_layer(y, d, ch, nbuf):
        h = y[:, :d]
        asrc = y[:, 128]
        adst = y[:, 129]
        ex, s = _sca(src, dst, asrc, adst)
        sc = _scomb(s)
        acc, alpha = _make_scb(d, ch, nbuf)(
            src16, dst16, ex.reshape(32, KT // ch, ch), sc, h)
        return acc, alpha.reshape(EPAD)

    y1 = _mm_first(x_pad, w1p, a1p)
    acc1, alpha = _edge_layer(y1, 128, 64, 2)
    y2 = _mm_mid(acc1, b1[None, :], w2p, a2p)
    acc2, _ = _edge_layer(y2, 128, 64, 2)
    y3 = _mm_mid(acc2, b2[None, :], w3p, a3p)
    acc3, _ = _edge_layer(y3, 64, 128, 3)
    h3 = _sum_final(acc3, b3[None, :])

    return (ei2, alpha[:E2], h3[:N])


# speedup vs baseline: 1.0459x; 1.0074x over previous
import functools

import jax
import jax.numpy as jnp
from jax import lax
from jax.experimental import pallas as pl
from jax.experimental.pallas import tpu as pltpu
from jax.experimental.pallas import tpu_sc as plsc

N = 10000
E = 320000
E2 = E + N
NPAD = 10112
EPAD = 331776
KT = EPAD // 32
NCH = KT // 128
RPT = NPAD // 16

_mesh = plsc.VectorSubcoreMesh(core_axis_name="c", subcore_axis_name="s")
_sc_params = pltpu.CompilerParams(needs_layout_passes=False,
                                  use_tc_tiling_on_sc=False)


def _iota16():
    return lax.iota(jnp.int32, 16)


def _mm_first_body(x_ref, w_ref, a_ref, o_ref):
    h = jnp.dot(x_ref[...], w_ref[...], preferred_element_type=jnp.float32)
    av = jnp.dot(h, a_ref[...], preferred_element_type=jnp.float32)
    o_ref[:, :128] = h
    o_ref[:, 128:] = av


def _mm_mid_body(p0_ref, p1_ref, b_ref, w_ref, a_ref, o_ref):
    x = jnp.maximum(p0_ref[0] + p1_ref[0] + b_ref[0], 0.0)
    h = jnp.dot(x, w_ref[...], preferred_element_type=jnp.float32)
    av = jnp.dot(h, a_ref[...], preferred_element_type=jnp.float32)
    o_ref[:, :128] = h
    o_ref[:, 128:] = av


def _sum_body(p0_ref, p1_ref, b_ref, o_ref):
    o_ref[...] = p0_ref[0] + p1_ref[0] + b_ref[0]


def _scomb_body(s_ref, o_ref):
    o_ref[...] = 1.0 / (s_ref[0] + s_ref[1] + 1e-16)


def _scomb(s2):
    return pl.pallas_call(
        _scomb_body,
        in_specs=[pl.BlockSpec((2, NPAD // 128, 128), lambda: (0, 0, 0))],
        out_specs=pl.BlockSpec((NPAD // 128, 128), lambda: (0, 0)),
        out_shape=jax.ShapeDtypeStruct((NPAD // 128, 128), jnp.float32),
    )(s2.reshape(2, NPAD // 128, 128)).reshape(NPAD)


def _mm_first(x_pad, w_pad, a_pad):
    return pl.pallas_call(
        _mm_first_body,
        grid=(NPAD // 128,),
        in_specs=[
            pl.BlockSpec((128, 128), lambda i: (i, 0)),
            pl.BlockSpec((128, 128), lambda i: (0, 0)),
            pl.BlockSpec((128, 128), lambda i: (0, 0)),
        ],
        out_specs=pl.BlockSpec((128, 256), lambda i: (i, 0)),
        out_shape=jax.ShapeDtypeStruct((NPAD, 256), jnp.float32),
    )(x_pad, w_pad, a_pad)


def _mm_mid(acc, b, w_pad, a_pad):
    return pl.pallas_call(
        _mm_mid_body,
        grid=(NPAD // 128,),
        in_specs=[
            pl.BlockSpec((1, 128, 128), lambda i: (0, i, 0)),
            pl.BlockSpec((1, 128, 128), lambda i: (1, i, 0)),
            pl.BlockSpec((1, 128), lambda i: (0, 0)),
            pl.BlockSpec((128, 128), lambda i: (0, 0)),
            pl.BlockSpec((128, 128), lambda i: (0, 0)),
        ],
        out_specs=pl.BlockSpec((128, 256), lambda i: (i, 0)),
        out_shape=jax.ShapeDtypeStruct((NPAD, 256), jnp.float32),
    )(acc, acc, b, w_pad, a_pad)


def _sum_final(acc, b):
    d = acc.shape[-1]
    return pl.pallas_call(
        _sum_body,
        grid=(NPAD // 128,),
        in_specs=[
            pl.BlockSpec((1, 128, d), lambda i: (0, i, 0)),
            pl.BlockSpec((1, 128, d), lambda i: (1, i, 0)),
            pl.BlockSpec((1, d), lambda i: (0, 0)),
        ],
        out_specs=pl.BlockSpec((128, d), lambda i: (i, 0)),
        out_shape=jax.ShapeDtypeStruct((NPAD, d), jnp.float32),
    )(acc, acc, b)


def _sca_body(src_hbm, dst_hbm, asrc_hbm, adst_hbm,
              ex_hbm, s_hbm,
              src_v, dst_v, asrc_v, adst_v, ex_v, rows0, rows1, dstg0,
              dstg1, ztile_v, sc_v, s2d_sh, sem, sem2):
    cid = lax.axis_index("c")
    sid = lax.axis_index("s")
    wid = cid * 16 + sid
    rows = (rows0, rows1)
    dstage = (dstg0, dstg1)
    sems = (sem, sem2)

    pltpu.sync_copy(src_hbm.at[wid], src_v)
    pltpu.sync_copy(dst_hbm.at[wid], dst_v)
    pltpu.sync_copy(asrc_hbm, asrc_v)
    pltpu.sync_copy(adst_hbm, adst_v)

    zv = jnp.zeros((16,), jnp.float32)

    def _zrow(i, _):
        ztile_v[i, :] = zv
        return 0

    lax.fori_loop(0, RPT, _zrow, 0)
    pltpu.sync_copy(ztile_v, s2d_sh.at[pl.ds(sid * RPT, RPT)])

    def _zrow2(i, _):
        rows0[i, :] = zv
        rows1[i, :] = zv
        return 0

    lax.fori_loop(0, 128, _zrow2, 0)
    plsc.subcore_barrier()

    iota = _iota16()
    zcol = jnp.zeros((16,), jnp.int32)

    def _compute(j, b):
        for g in range(8):
            sv = src_v[j, pl.ds(g * 16, 16)]
            dv = dst_v[j, pl.ds(g * 16, 16)]
            e = plsc.load_gather(asrc_v, [sv]) + plsc.load_gather(adst_v, [dv])
            e = jnp.where(e >= 0.0, e, 0.2 * e)
            exv = jnp.exp(e)
            ex_v[j, pl.ds(g * 16, 16)] = exv
            plsc.store_scatter(rows[b], [iota + (g * 16), zcol], exv)
            dstage[b][pl.ds(g * 16, 16)] = dv

    def _chunk(jj, _):
        for b in range(2):
            j = jj * 2 + b

            @pl.when(jj > 0)
            def _():
                pltpu.make_async_copy(rows[b], s2d_sh.at[dstage[b]],
                                      sems[b]).wait()

            _compute(j, b)
            pltpu.async_copy(rows[b], s2d_sh.at[dstage[b]], sems[b],
                             add=True)
        return 0

    lax.fori_loop(0, NCH // 2, _chunk, 0)
    pltpu.make_async_copy(rows[0], s2d_sh.at[dstage[0]], sem).wait()
    _compute(NCH - 1, 0)
    pltpu.async_copy(rows[0], s2d_sh.at[dstage[0]], sem, add=True)
    pltpu.make_async_copy(rows[0], s2d_sh.at[dstage[0]], sem).wait()
    pltpu.make_async_copy(rows[1], s2d_sh.at[dstage[1]], sem2).wait()
    plsc.subcore_barrier()

    pltpu.sync_copy(ex_v, ex_hbm.at[wid])

    pltpu.sync_copy(s2d_sh.at[pl.ds(sid * RPT, RPT)], ztile_v)

    def _comp(i, _):
        base = jnp.minimum(i * 16, RPT - 16)
        sc_v[pl.ds(base, 16)] = plsc.load_gather(ztile_v, [iota + base, zcol])
        return 0

    lax.fori_loop(0, (RPT + 15) // 16, _comp, 0)
    pltpu.sync_copy(sc_v, s_hbm.at[cid, pl.ds(sid * RPT, RPT)])


_sca = pl.kernel(
    _sca_body,
    out_type=(
        jax.ShapeDtypeStruct((32, NCH, 128), jnp.float32),
        jax.ShapeDtypeStruct((2, NPAD), jnp.float32),
    ),
    mesh=_mesh,
    compiler_params=_sc_params,
    scratch_types=(
        pltpu.VMEM((NCH, 128), jnp.int32),
        pltpu.VMEM((NCH, 128), jnp.int32),
        pltpu.VMEM((NPAD,), jnp.float32),
        pltpu.VMEM((NPAD,), jnp.float32),
        pltpu.VMEM((NCH, 128), jnp.float32),
        pltpu.VMEM((128, 16), jnp.float32),
        pltpu.VMEM((128, 16), jnp.float32),
        pltpu.VMEM((128,), jnp.int32),
        pltpu.VMEM((128,), jnp.int32),
        pltpu.VMEM((RPT, 16), jnp.float32),
        pltpu.VMEM((RPT,), jnp.float32),
        pltpu.VMEM_SHARED((NPAD, 16), jnp.float32),
        pltpu.SemaphoreType.DMA,
        pltpu.SemaphoreType.DMA,
    ),
)


def _scb_body(d, ch, nbuf, src16_hbm, dst16_hbm, ex_hbm, s_hbm, h_hbm,
              acc_hbm, al_hbm, *scr):
    nc2 = KT // ch
    sb_v, db_v, al_v, s_v = scr[:4]
    rows = scr[4:4 + nbuf]
    ssts = scr[4 + nbuf:4 + 2 * nbuf]
    dsts = scr[4 + 2 * nbuf:4 + 3 * nbuf]
    acc_sh = scr[4 + 3 * nbuf]
    sgs = scr[5 + 3 * nbuf:5 + 4 * nbuf]
    sss = scr[5 + 4 * nbuf:5 + 5 * nbuf]
    cid = lax.axis_index("c")
    sid = lax.axis_index("s")
    wid = cid * 16 + sid
    nk = d // 16

    pltpu.sync_copy(src16_hbm.at[wid], sb_v)
    pltpu.sync_copy(dst16_hbm.at[wid], db_v)
    pltpu.sync_copy(ex_hbm.at[wid], al_v)
    pltpu.sync_copy(s_hbm, s_v)

    zv = jnp.zeros((16,), jnp.float32)

    def _zrow(i, _):
        for k in range(nk):
            rows[0][i, pl.ds(k * 16, 16)] = zv
        return 0

    lax.fori_loop(0, ch, _zrow, 0)
    for r0 in range(0, RPT, ch):
        rr = min(ch, RPT - r0)
        pltpu.sync_copy(rows[0].at[pl.ds(0, rr)],
                        acc_sh.at[pl.ds(sid * RPT + r0, rr)])
    plsc.subcore_barrier()

    iota = _iota16()
    mlo = jnp.full((16,), 0xFFFF, jnp.int32)

    def _unpack(j, packed_ref, stage):
        for g in range(ch // 32):
            w = packed_ref[pl.ds(j * (ch // 2) + g * 16, 16)]
            plsc.store_scatter(stage, [iota * 2 + g * 32],
                               jnp.bitwise_and(w, mlo))
            plsc.store_scatter(stage, [iota * 2 + g * 32 + 1],
                               lax.shift_right_logical(w, 16))

    def _iter(jj, _):
        descs = []
        for b in range(nbuf):
            j = jj * nbuf + b

            @pl.when(jj > 0)
            def _():
                pltpu.make_async_copy(rows[b], acc_sh.at[dsts[b]],
                                      sss[b]).wait()

            _unpack(j, sb_v, ssts[b])
            _unpack(j, db_v, dsts[b])
            descs.append(pltpu.async_copy(h_hbm.at[ssts[b]], rows[b],
                                          sgs[b]))
        for b in range(nbuf):
            j = jj * nbuf + b

            def _div(g, _):
                sl = pl.ds(g * 16, 16)
                rsv = plsc.load_gather(s_v, [dsts[b][sl]])
                al_v[j, sl] = al_v[j, sl] * rsv
                return 0

            lax.fori_loop(0, ch // 16, _div, 0)
            descs[b].wait()

            jv = jnp.full((16,), j, jnp.int32)

            def _grp(g, _):
                base = g * 16

                def _one(e16):
                    e = base + e16
                    splat = plsc.load_gather(
                        al_v, [jv, jnp.full((16,), e16, jnp.int32) + base])
                    for k in range(nk):
                        rows[b][e, pl.ds(k * 16, 16)] = (
                            rows[b][e, pl.ds(k * 16, 16)] * splat)

                for e16 in range(16):
                    _one(e16)
                return 0

            lax.fori_loop(0, ch // 16, _grp, 0)
            pltpu.async_copy(rows[b], acc_sh.at[dsts[b]], sss[b], add=True)
        return 0

    lax.fori_loop(0, nc2 // nbuf, _iter, 0)
    for b in range(nbuf):
        pltpu.make_async_copy(rows[b], acc_sh.at[dsts[b]], sss[b]).wait()
    plsc.subcore_barrier()

    pltpu.sync_copy(acc_sh.at[pl.ds(sid * RPT, RPT)],
                    acc_hbm.at[cid, pl.ds(sid * RPT, RPT)])
    pltpu.sync_copy(al_v, al_hbm.at[wid])


@functools.cache
def _make_scb(d, ch, nbuf):
    nc2 = KT // ch
    return pl.kernel(
        functools.partial(_scb_body, d, ch, nbuf),
        out_type=(
            jax.ShapeDtypeStruct((2, NPAD, d), jnp.float32),
            jax.ShapeDtypeStruct((32, nc2, ch), jnp.float32),
        ),
        mesh=_mesh,
        compiler_params=_sc_params,
        scratch_types=(
            pltpu.VMEM((KT // 2,), jnp.int32),
            pltpu.VMEM((KT // 2,), jnp.int32),
            pltpu.VMEM((nc2, ch), jnp.float32),
            pltpu.VMEM((NPAD,), jnp.float32),
            *[pltpu.VMEM((ch, d), jnp.float32) for _ in range(nbuf)],
            *[pltpu.VMEM((ch,), jnp.int32) for _ in range(2 * nbuf)],
            pltpu.VMEM_SHARED((NPAD, d), jnp.float32),
            *[pltpu.SemaphoreType.DMA for _ in range(2 * nbuf)],
        ),
    )


def _pack_attn(a_s, a_d):
    a = jnp.stack([a_s, a_d], axis=1)
    return jnp.pad(a, ((0, 128 - a.shape[0]), (0, 126)))


def kernel(x, edge_index, W1, as1, ad1, b1, W2, as2, ad2, b2,
           W3, as3, ad3, b3):
    loops = jnp.arange(N, dtype=edge_index.dtype)
    ei2 = jnp.concatenate(
        [edge_index, jnp.stack([loops, loops], axis=0)], axis=1)

    pad_s = jnp.full((EPAD - E2,), N, jnp.int32)
    pad_d = N + jnp.arange(EPAD - E2, dtype=jnp.int32) % (NPAD - N)
    src_f = jnp.concatenate([ei2[0], pad_s])
    dst_f = jnp.concatenate([ei2[1], pad_d])
    src = src_f.reshape(32, NCH, 128)
    dst = dst_f.reshape(32, NCH, 128)

    def _pack16(a):
        return lax.bitcast_convert_type(
            a.astype(jnp.int16).reshape(EPAD // 2, 2),
            jnp.int32).reshape(32, KT // 2)

    src16 = _pack16(src_f)
    dst16 = _pack16(dst_f)

    x_pad = jnp.pad(x, ((0, NPAD - N), (0, 0)))
    w1p = W1
    w2p = W2
    w3p = jnp.pad(W3, ((0, 0), (0, 128 - W3.shape[1])))
    a1p = _pack_attn(as1, ad1)
    a2p = _pack_attn(as2, ad2)
    a3p = _pack_attn(as3, ad3)

    def _edge_layer(y, d, ch, nbuf):
        h = y[:, :d]
        asrc = y[:, 128]
        adst = y[:, 129]
        ex, s = _sca(src, dst, asrc, adst)
        sc = _scomb(s)
        acc, alpha = _make_scb(d, ch, nbuf)(
            src16, dst16, ex.reshape(32, KT // ch, ch), sc, h)
        return acc, alpha.reshape(EPAD)

    y1 = _mm_first(x_pad, w1p, a1p)
    acc1, alpha = _edge_layer(y1, 128, 64, 2)
    y2 = _mm_mid(acc1, b1[None, :], w2p, a2p)
    acc2, _ = _edge_layer(y2, 128, 64, 2)
    y3 = _mm_mid(acc2, b2[None, :], w3p, a3p)
    acc3, _ = _edge_layer(y3, 64, 128, 3)
    h3 = _sum_final(acc3, b3[None, :])

    return (ei2, alpha[:E2], h3[:N])
